# baseline (speedup 1.0000x reference)
import functools

import jax
import jax.numpy as jnp
from jax import lax
from jax.experimental import pallas as pl
from jax.experimental.pallas import tpu as pltpu
from jax.experimental.pallas import tpu_sc as plsc

MAX_LEN = 200
EMBED_DIM = 64
BATCH = 16384

_info = plsc.get_sparse_core_info()
_NC, _NS, _L = _info.num_cores, _info.num_subcores, _info.num_lanes
_NW = _NC * _NS
_B_PER_W = BATCH // _NW
_CHUNK = 128
_N_CHUNKS = _B_PER_W // _CHUNK


def _gather_body(table_hbm, match_hbm, prefix_hbm, out_hbm,
                 match_v, prefix_v, idx_v, rows_v, sem):
    wid = lax.axis_index("s") * _NC + lax.axis_index("c")
    base = wid * _B_PER_W

    pltpu.sync_copy(match_hbm.at[pl.ds(base, _B_PER_W)], match_v)
    pltpu.sync_copy(prefix_hbm.at[pl.ds(base, _B_PER_W)], prefix_v)

    for i in range(_B_PER_W // _L):
        sl = pl.ds(i * _L, _L)
        idx_v[sl] = match_v[sl] * MAX_LEN + prefix_v[sl]

    copies = []
    for j in range(_N_CHUNKS):
        sl = pl.ds(j * _CHUNK, _CHUNK)
        copies.append(pltpu.async_copy(table_hbm.at[idx_v.at[sl]],
                                       rows_v.at[sl], sem))
    for c in copies:
        c.wait()

    pltpu.sync_copy(rows_v, out_hbm.at[pl.ds(base, _B_PER_W)])


@jax.jit
def _gather(flat_table, match_idx, prefix_idx):
    mesh = plsc.VectorSubcoreMesh(core_axis_name="c", subcore_axis_name="s")
    return pl.kernel(
        _gather_body,
        mesh=mesh,
        out_type=jax.ShapeDtypeStruct((BATCH, EMBED_DIM), jnp.float32),
        scratch_types=[
            pltpu.VMEM((_B_PER_W,), jnp.int32),
            pltpu.VMEM((_B_PER_W,), jnp.int32),
            pltpu.VMEM((_B_PER_W,), jnp.int32),
            pltpu.VMEM((_B_PER_W, EMBED_DIM), jnp.float32),
            pltpu.SemaphoreType.DMA,
        ],
        compiler_params=pltpu.CompilerParams(use_tc_tiling_on_sc=False),
    )(flat_table, match_idx, prefix_idx)


def kernel(table, match_len_idx, prefix_len_idx):
    flat_table = table.reshape(MAX_LEN * MAX_LEN, EMBED_DIM)
    return _gather(flat_table,
                   match_len_idx.astype(jnp.int32),
                   prefix_len_idx.astype(jnp.int32))
